# Initial kernel scaffold; baseline (speedup 1.0000x reference)
import functools

import jax
import jax.numpy as jnp
from jax import lax
from jax.experimental import pallas as pl
from jax.experimental.pallas import tpu as pltpu
from jax.experimental.pallas import tpu_sc as plsc

N = 10000
E = 320000
H = 256
HH = 128
NC, NS = 2, 16
NPAD = 10240
STRIPE = NPAD // NS

F32 = jnp.float32

_mesh = plsc.VectorSubcoreMesh(core_axis_name="c", subcore_axis_name="s")


@functools.partial(
    pl.kernel,
    out_type=jax.ShapeDtypeStruct((NC * NPAD,), F32),
    mesh=_mesh,
    scratch_types=[
        pltpu.VMEM((128,), jnp.int32),
        pltpu.VMEM((128,), F32),
        pltpu.VMEM((16,), jnp.int32),
        pltpu.VMEM((16,), F32),
        pltpu.VMEM((STRIPE,), F32),
        pltpu.VMEM_SHARED((NPAD,), F32),
    ],
)
def _deg_kernel(dst_hbm, out_hbm, dstb, onesb, dstb_t, onesb_t, stage, degsp):
    c = lax.axis_index("c")
    s = lax.axis_index("s")
    for j in range(8):
        onesb[pl.ds(16 * j, 16)] = jnp.full((16,), 1.0, F32)
    onesb_t[pl.ds(0, 16)] = jnp.full((16,), 1.0, F32)
    for j in range(STRIPE // 16):
        stage[pl.ds(16 * j, 16)] = jnp.zeros((16,), F32)
    pltpu.sync_copy(stage, degsp.at[pl.ds(s * STRIPE, STRIPE)])
    plsc.subcore_barrier()

    per_tile = E // (NC * NS)
    base = (s * NC + c) * per_tile
    nfull = per_tile // 128
    tail = per_tile - nfull * 128

    def chunk(k, carry):
        b = base + k * 128
        pltpu.sync_copy(dst_hbm.at[pl.ds(b, 128)], dstb)
        pltpu.sync_copy(onesb, degsp.at[dstb], add=True)
        return carry

    lax.fori_loop(0, nfull, chunk, 0)
    bt = base + nfull * 128
    pltpu.sync_copy(dst_hbm.at[pl.ds(bt, tail)], dstb_t)
    pltpu.sync_copy(onesb_t, degsp.at[dstb_t], add=True)
    plsc.subcore_barrier()

    pltpu.sync_copy(degsp.at[pl.ds(s * STRIPE, STRIPE)], stage)
    pltpu.sync_copy(stage, out_hbm.at[pl.ds(c * NPAD + s * STRIPE, STRIPE)])


@functools.partial(
    pl.kernel,
    out_type=jax.ShapeDtypeStruct((NC * N, HH), F32),
    mesh=_mesh,
    scratch_types=[
        pltpu.VMEM((128,), jnp.int32),
        pltpu.VMEM((128,), jnp.int32),
        pltpu.VMEM((128,), jnp.int32),
        pltpu.VMEM((128, HH), F32),
        pltpu.VMEM((128, HH), F32),
        pltpu.VMEM((32,), jnp.int32),
        pltpu.VMEM((32,), jnp.int32),
        pltpu.VMEM((32,), jnp.int32),
        pltpu.VMEM((32, HH), F32),
        pltpu.VMEM_SHARED((NPAD, HH), F32),
        pltpu.SemaphoreType.DMA,
    ],
)
def _agg_kernel(hs_hbm, src_hbm, dst_hbm, out_hbm,
                srcb, dstb, idxb, rows, zb,
                srcb_t, dstb_t, idxb_t, rows_t, aggsp, sem):
    c = lax.axis_index("c")
    s = lax.axis_index("s")
    rowoff = c * N

    def zbody(i, carry):
        for j in range(HH // 16):
            zb[i, pl.ds(16 * j, 16)] = jnp.zeros((16,), F32)
        return carry

    lax.fori_loop(0, 128, zbody, 0)
    for b5 in range(STRIPE // 128):
        pltpu.sync_copy(zb, aggsp.at[pl.ds(s * STRIPE + b5 * 128, 128)])
    plsc.subcore_barrier()

    per_tile = E // NS
    ebase = s * per_tile
    nfull = per_tile // 128
    tail = per_tile - nfull * 128

    def chunk(k, carry):
        b = ebase + k * 128
        pltpu.sync_copy(src_hbm.at[pl.ds(b, 128)], srcb)
        pltpu.sync_copy(dst_hbm.at[pl.ds(b, 128)], dstb)
        for j in range(8):
            idxb[pl.ds(16 * j, 16)] = srcb[pl.ds(16 * j, 16)] + rowoff
        pltpu.async_copy(hs_hbm.at[idxb], rows, sem).wait()
        pltpu.sync_copy(rows, aggsp.at[dstb], add=True)
        return carry

    lax.fori_loop(0, nfull, chunk, 0)
    bt = ebase + nfull * 128
    pltpu.sync_copy(src_hbm.at[pl.ds(bt, tail)], srcb_t)
    pltpu.sync_copy(dst_hbm.at[pl.ds(bt, tail)], dstb_t)
    for j in range(tail // 16):
        idxb_t[pl.ds(16 * j, 16)] = srcb_t[pl.ds(16 * j, 16)] + rowoff
    pltpu.async_copy(hs_hbm.at[idxb_t], rows_t, sem).wait()
    pltpu.sync_copy(rows_t, aggsp.at[dstb_t], add=True)
    plsc.subcore_barrier()

    @pl.when(s < NS - 1)
    def _():
        for b5 in range(STRIPE // 128):
            pltpu.sync_copy(aggsp.at[pl.ds(s * STRIPE + b5 * 128, 128)], zb)
            pltpu.sync_copy(zb, out_hbm.at[pl.ds(rowoff + s * STRIPE + b5 * 128, 128)])

    @pl.when(s == NS - 1)
    def _():
        lastbase = (NS - 1) * STRIPE
        for b5 in range(3):
            pltpu.sync_copy(aggsp.at[pl.ds(lastbase + b5 * 128, 128)], zb)
            pltpu.sync_copy(zb, out_hbm.at[pl.ds(rowoff + lastbase + b5 * 128, 128)])
        pltpu.sync_copy(aggsp.at[pl.ds(lastbase + 384, 16)], rows_t.at[pl.ds(0, 16)])
        pltpu.sync_copy(rows_t.at[pl.ds(0, 16)], out_hbm.at[pl.ds(rowoff + lastbase + 384, 16)])


@functools.partial(
    pl.kernel,
    out_type=jax.ShapeDtypeStruct((E,), F32),
    mesh=_mesh,
    scratch_types=[
        pltpu.VMEM((128,), jnp.int32),
        pltpu.VMEM((128,), jnp.int32),
        pltpu.VMEM((128, H), F32),
        pltpu.VMEM((128, H), F32),
        pltpu.VMEM((128,), F32),
        pltpu.VMEM((H,), F32),
        pltpu.VMEM((H,), F32),
        pltpu.VMEM((16,), F32),
        pltpu.VMEM((16,), jnp.int32),
        pltpu.VMEM((16,), jnp.int32),
        pltpu.VMEM((16, H), F32),
        pltpu.VMEM((16, H), F32),
        pltpu.VMEM((16,), F32),
        pltpu.SemaphoreType.DMA,
        pltpu.SemaphoreType.DMA,
    ],
)
def _edge_kernel(a_hbm, b_hbm, src_hbm, dst_hbm, bm1_hbm, wm2_hbm, bm2_hbm,
                 out_hbm, srcb, dstb, arows, brows, obuf, bmb, wmb, b2b,
                 srcb_t, dstb_t, arows_t, brows_t, obuf_t, sem1, sem2):
    c = lax.axis_index("c")
    s = lax.axis_index("s")
    pltpu.sync_copy(bm1_hbm, bmb)
    pltpu.sync_copy(wm2_hbm, wmb)
    pltpu.sync_copy(bm2_hbm, b2b)
    bmv = [bmb[pl.ds(16 * j, 16)] for j in range(H // 16)]
    wmv = [wmb[pl.ds(16 * j, 16)] for j in range(H // 16)]
    b2s = b2b[0]

    per_tile = E // (NC * NS)
    base = (s * NC + c) * per_tile
    nfull = per_tile // 128
    tail = per_tile - nfull * 128

    def do_chunk(b, nb, sb, db, ar, br_, ob):
        pltpu.sync_copy(src_hbm.at[pl.ds(b, nb)], sb)
        pltpu.sync_copy(dst_hbm.at[pl.ds(b, nb)], db)
        cp1 = pltpu.async_copy(a_hbm.at[sb], ar, sem1)
        cp2 = pltpu.async_copy(b_hbm.at[db], br_, sem2)
        cp1.wait()
        cp2.wait()

        def ebody(e, carry):
            acc = jnp.zeros((16,), F32)
            for j in range(H // 16):
                va = ar[e, pl.ds(16 * j, 16)]
                vb = br_[e, pl.ds(16 * j, 16)]
                acc = acc + jnp.maximum(va + vb + bmv[j], 0.0) * wmv[j]
            ob[e] = jnp.sum(acc) + b2s
            return carry

        lax.fori_loop(0, nb, ebody, 0)
        pltpu.sync_copy(ob, out_hbm.at[pl.ds(b, nb)])

    def chunk(k, carry):
        do_chunk(base + k * 128, 128, srcb, dstb, arows, brows, obuf)
        return carry

    lax.fori_loop(0, nfull, chunk, 0)
    do_chunk(base + nfull * 128, tail, srcb_t, dstb_t, arows_t, brows_t, obuf_t)


RB = 1000
GRID = N // RB


def _dinv_block(dpr):
    deg = dpr[0] + dpr[1] + 1.0
    return lax.rsqrt(jnp.maximum(deg, 1.0))


def _mm_scale_body(xr, wr, br, dpr, outr):
    dinv = _dinv_block(dpr)
    h = jnp.dot(xr[...], wr[...], preferred_element_type=F32) + br[...]
    hs = h * dinv
    outr[0] = hs[:, :HH]
    outr[1] = hs[:, HH:]


def _mm_scale(x, W, b, degp, fin):
    return pl.pallas_call(
        _mm_scale_body,
        grid=(GRID,),
        in_specs=[
            pl.BlockSpec((RB, fin), lambda i: (i, 0)),
            pl.BlockSpec((fin, H), lambda i: (0, 0)),
            pl.BlockSpec((1, H), lambda i: (0, 0)),
            pl.BlockSpec((2, RB, 1), lambda i: (0, i, 0)),
        ],
        out_specs=pl.BlockSpec((2, RB, HH), lambda i: (0, i, 0)),
        out_shape=jax.ShapeDtypeStruct((2, N, HH), F32),
    )(x, W, b, degp)


def _agg_block(aggr, hsr, dpr):
    dinv = _dinv_block(dpr)
    a0 = (aggr[0] + hsr[0]) * dinv
    a1 = (aggr[1] + hsr[1]) * dinv
    return jnp.concatenate([a0, a1], axis=1)


def _stats_body(aggr, hsr, dpr, outr):
    i = pl.program_id(0)
    a = _agg_block(aggr, hsr, dpr)
    blk = jnp.stack([jnp.sum(a, axis=0), jnp.sum(a * a, axis=0)])

    @pl.when(i == 0)
    def _():
        outr[...] = jnp.zeros((2, H), F32)

    outr[...] += blk


def _stats(agg0, hs, degp):
    return pl.pallas_call(
        _stats_body,
        grid=(GRID,),
        in_specs=[
            pl.BlockSpec((2, RB, HH), lambda i: (0, i, 0)),
            pl.BlockSpec((2, RB, HH), lambda i: (0, i, 0)),
            pl.BlockSpec((2, RB, 1), lambda i: (0, i, 0)),
        ],
        out_specs=pl.BlockSpec((2, H), lambda i: (0, 0)),
        out_shape=jax.ShapeDtypeStruct((2, H), F32),
    )(agg0, hs, degp)


def _bn_relu(aggr, hsr, dpr, str_, gr, btr):
    a = _agg_block(aggr, hsr, dpr)
    mean = str_[0] * (1.0 / N)
    var = str_[1] * (1.0 / N) - mean * mean
    xn = gr[...] * (a - mean) * lax.rsqrt(var + 1e-5) + btr[...]
    return jnp.maximum(xn, 0.0)


def _bn_mm_scale_body(aggr, hsr, dpr, str_, gr, btr, wr, br, outr):
    o = _bn_relu(aggr, hsr, dpr, str_, gr, btr)
    h2 = jnp.dot(o, wr[...], preferred_element_type=F32) + br[...]
    hs2 = h2 * _dinv_block(dpr)
    outr[0] = hs2[:, :HH]
    outr[1] = hs2[:, HH:]


def _bn_mm_scale(agg0, hs, degp, stats, g, bt, W, b):
    return pl.pallas_call(
        _bn_mm_scale_body,
        grid=(GRID,),
        in_specs=[
            pl.BlockSpec((2, RB, HH), lambda i: (0, i, 0)),
            pl.BlockSpec((2, RB, HH), lambda i: (0, i, 0)),
            pl.BlockSpec((2, RB, 1), lambda i: (0, i, 0)),
            pl.BlockSpec((2, H), lambda i: (0, 0)),
            pl.BlockSpec((1, H), lambda i: (0, 0)),
            pl.BlockSpec((1, H), lambda i: (0, 0)),
            pl.BlockSpec((H, H), lambda i: (0, 0)),
            pl.BlockSpec((1, H), lambda i: (0, 0)),
        ],
        out_specs=pl.BlockSpec((2, RB, HH), lambda i: (0, i, 0)),
        out_shape=jax.ShapeDtypeStruct((2, N, HH), F32),
    )(agg0, hs, degp, stats, g, bt, W, b)


def _bn_ab_body(aggr, hsr, dpr, str_, gr, btr, war, wbr, outa, outb):
    h2 = _bn_relu(aggr, hsr, dpr, str_, gr, btr)
    outa[...] = jnp.dot(h2, war[...], preferred_element_type=F32)
    outb[...] = jnp.dot(h2, wbr[...], preferred_element_type=F32)


def _bn_ab(agg0, hs, degp, stats, g, bt, Wa, Wb):
    return pl.pallas_call(
        _bn_ab_body,
        grid=(GRID,),
        in_specs=[
            pl.BlockSpec((2, RB, HH), lambda i: (0, i, 0)),
            pl.BlockSpec((2, RB, HH), lambda i: (0, i, 0)),
            pl.BlockSpec((2, RB, 1), lambda i: (0, i, 0)),
            pl.BlockSpec((2, H), lambda i: (0, 0)),
            pl.BlockSpec((1, H), lambda i: (0, 0)),
            pl.BlockSpec((1, H), lambda i: (0, 0)),
            pl.BlockSpec((H, H), lambda i: (0, 0)),
            pl.BlockSpec((H, H), lambda i: (0, 0)),
        ],
        out_specs=[
            pl.BlockSpec((RB, H), lambda i: (i, 0)),
            pl.BlockSpec((RB, H), lambda i: (i, 0)),
        ],
        out_shape=[
            jax.ShapeDtypeStruct((N, H), F32),
            jax.ShapeDtypeStruct((N, H), F32),
        ],
    )(agg0, hs, degp, stats, g, bt, Wa, Wb)


def kernel(x, edge_index, W1, b1, g1, bt1, W2, b2, g2, bt2, Wm1, bm1, Wm2, bm2):
    src = edge_index[0]
    dst = edge_index[1]
    b1r, g1r, bt1r = b1[None, :], g1[None, :], bt1[None, :]
    b2r, g2r, bt2r = b2[None, :], g2[None, :], bt2[None, :]

    degf = _deg_kernel(dst)
    degp = degf.reshape(NC, NPAD)[:, :N].reshape(NC, N, 1)

    hs1 = _mm_scale(x, W1, b1r, degp, 128)
    agg1 = _agg_kernel(hs1.reshape(NC * N, HH), src, dst)
    agg1 = agg1.reshape(NC, N, HH)
    st1 = _stats(agg1, hs1, degp)
    hs2 = _bn_mm_scale(agg1, hs1, degp, st1, g1r, bt1r, W2, b2r)

    agg2 = _agg_kernel(hs2.reshape(NC * N, HH), src, dst)
    agg2 = agg2.reshape(NC, N, HH)
    st2 = _stats(agg2, hs2, degp)
    A, B = _bn_ab(agg2, hs2, degp, st2, g2r, bt2r, Wm1[:H], Wm1[H:])

    wm2 = Wm2[:, 0]
    bm2b = jnp.broadcast_to(bm2, (16,))
    out = _edge_kernel(A, B, src, dst, bm1, wm2, bm2b)
    return out.reshape(E, 1)


# speedup vs baseline: 6.7684x; 6.7684x over previous
import functools

import jax
import jax.numpy as jnp
from jax import lax
from jax.experimental import pallas as pl
from jax.experimental.pallas import tpu as pltpu
from jax.experimental.pallas import tpu_sc as plsc

N = 10000
E = 320000
H = 256
HH = 128
NC, NS = 2, 16
NPAD = 10240
STRIPE = NPAD // NS

F32 = jnp.float32

_mesh = plsc.VectorSubcoreMesh(core_axis_name="c", subcore_axis_name="s")


@functools.partial(
    pl.kernel,
    out_type=jax.ShapeDtypeStruct((NC * NPAD,), F32),
    mesh=_mesh,
    scratch_types=[
        pltpu.VMEM((128,), jnp.int32),
        pltpu.VMEM((128,), F32),
        pltpu.VMEM((16,), jnp.int32),
        pltpu.VMEM((16,), F32),
        pltpu.VMEM((STRIPE,), F32),
        pltpu.VMEM_SHARED((NPAD,), F32),
    ],
)
def _deg_kernel(dst_hbm, out_hbm, dstb, onesb, dstb_t, onesb_t, stage, degsp):
    c = lax.axis_index("c")
    s = lax.axis_index("s")
    for j in range(8):
        onesb[pl.ds(16 * j, 16)] = jnp.full((16,), 1.0, F32)
    onesb_t[pl.ds(0, 16)] = jnp.full((16,), 1.0, F32)
    for j in range(STRIPE // 16):
        stage[pl.ds(16 * j, 16)] = jnp.zeros((16,), F32)
    pltpu.sync_copy(stage, degsp.at[pl.ds(s * STRIPE, STRIPE)])
    plsc.subcore_barrier()

    per_tile = E // (NC * NS)
    base = (s * NC + c) * per_tile
    nfull = per_tile // 128
    tail = per_tile - nfull * 128

    def chunk(k, carry):
        b = base + k * 128
        pltpu.sync_copy(dst_hbm.at[pl.ds(b, 128)], dstb)
        pltpu.sync_copy(onesb, degsp.at[dstb], add=True)
        return carry

    lax.fori_loop(0, nfull, chunk, 0)
    bt = base + nfull * 128
    pltpu.sync_copy(dst_hbm.at[pl.ds(bt, tail)], dstb_t)
    pltpu.sync_copy(onesb_t, degsp.at[dstb_t], add=True)
    plsc.subcore_barrier()

    pltpu.sync_copy(degsp.at[pl.ds(s * STRIPE, STRIPE)], stage)
    pltpu.sync_copy(stage, out_hbm.at[pl.ds(c * NPAD + s * STRIPE, STRIPE)])


@functools.partial(
    pl.kernel,
    out_type=jax.ShapeDtypeStruct((NC * N, HH), F32),
    mesh=_mesh,
    scratch_types=[
        pltpu.VMEM((128,), jnp.int32),
        pltpu.VMEM((128,), jnp.int32),
        pltpu.VMEM((128,), jnp.int32),
        pltpu.VMEM((128, HH), F32),
        pltpu.VMEM((128, HH), F32),
        pltpu.VMEM((32,), jnp.int32),
        pltpu.VMEM((32,), jnp.int32),
        pltpu.VMEM((32,), jnp.int32),
        pltpu.VMEM((32, HH), F32),
        pltpu.VMEM_SHARED((NPAD, HH), F32),
        pltpu.SemaphoreType.DMA,
    ],
)
def _agg_kernel(hs_hbm, src_hbm, dst_hbm, out_hbm,
                srcb, dstb, idxb, rows, zb,
                srcb_t, dstb_t, idxb_t, rows_t, aggsp, sem):
    c = lax.axis_index("c")
    s = lax.axis_index("s")
    rowoff = c * N

    def zbody(i, carry):
        for j in range(HH // 16):
            zb[i, pl.ds(16 * j, 16)] = jnp.zeros((16,), F32)
        return carry

    lax.fori_loop(0, 128, zbody, 0)
    for b5 in range(STRIPE // 128):
        pltpu.sync_copy(zb, aggsp.at[pl.ds(s * STRIPE + b5 * 128, 128)])
    plsc.subcore_barrier()

    per_tile = E // NS
    ebase = s * per_tile
    nfull = per_tile // 128
    tail = per_tile - nfull * 128

    def chunk(k, carry):
        b = ebase + k * 128
        pltpu.sync_copy(src_hbm.at[pl.ds(b, 128)], srcb)
        pltpu.sync_copy(dst_hbm.at[pl.ds(b, 128)], dstb)
        for j in range(8):
            idxb[pl.ds(16 * j, 16)] = srcb[pl.ds(16 * j, 16)] + rowoff
        pltpu.async_copy(hs_hbm.at[idxb], rows, sem).wait()
        pltpu.sync_copy(rows, aggsp.at[dstb], add=True)
        return carry

    lax.fori_loop(0, nfull, chunk, 0)
    bt = ebase + nfull * 128
    pltpu.sync_copy(src_hbm.at[pl.ds(bt, tail)], srcb_t)
    pltpu.sync_copy(dst_hbm.at[pl.ds(bt, tail)], dstb_t)
    for j in range(tail // 16):
        idxb_t[pl.ds(16 * j, 16)] = srcb_t[pl.ds(16 * j, 16)] + rowoff
    pltpu.async_copy(hs_hbm.at[idxb_t], rows_t, sem).wait()
    pltpu.sync_copy(rows_t, aggsp.at[dstb_t], add=True)
    plsc.subcore_barrier()

    @pl.when(s < NS - 1)
    def _():
        for b5 in range(STRIPE // 128):
            pltpu.sync_copy(aggsp.at[pl.ds(s * STRIPE + b5 * 128, 128)], zb)
            pltpu.sync_copy(zb, out_hbm.at[pl.ds(rowoff + s * STRIPE + b5 * 128, 128)])

    @pl.when(s == NS - 1)
    def _():
        lastbase = (NS - 1) * STRIPE
        for b5 in range(3):
            pltpu.sync_copy(aggsp.at[pl.ds(lastbase + b5 * 128, 128)], zb)
            pltpu.sync_copy(zb, out_hbm.at[pl.ds(rowoff + lastbase + b5 * 128, 128)])
        pltpu.sync_copy(aggsp.at[pl.ds(lastbase + 384, 16)], rows_t.at[pl.ds(0, 16)])
        pltpu.sync_copy(rows_t.at[pl.ds(0, 16)], out_hbm.at[pl.ds(rowoff + lastbase + 384, 16)])


@functools.partial(
    pl.kernel,
    out_type=jax.ShapeDtypeStruct((E,), F32),
    mesh=_mesh,
    scratch_types=[
        pltpu.VMEM((128,), jnp.int32),
        pltpu.VMEM((128,), jnp.int32),
        pltpu.VMEM((128, H), F32),
        pltpu.VMEM((128, H), F32),
        pltpu.VMEM((128,), F32),
        pltpu.VMEM((H,), F32),
        pltpu.VMEM((H,), F32),
        pltpu.VMEM((16,), F32),
        pltpu.VMEM((16,), jnp.int32),
        pltpu.VMEM((16,), jnp.int32),
        pltpu.VMEM((16, H), F32),
        pltpu.VMEM((16, H), F32),
        pltpu.VMEM((16,), F32),
        pltpu.VMEM((32,), F32),
        pltpu.SemaphoreType.DMA,
        pltpu.SemaphoreType.DMA,
    ],
)
def _edge_kernel(a_hbm, b_hbm, src_hbm, dst_hbm, bm1_hbm, wm2_hbm, bm2_hbm,
                 out_hbm, srcb, dstb, arows, brows, obuf, bmb, wmb, b2b,
                 srcb_t, dstb_t, arows_t, brows_t, obuf_t, tbuf, sem1, sem2):
    c = lax.axis_index("c")
    s = lax.axis_index("s")
    pltpu.sync_copy(bm1_hbm, bmb)
    pltpu.sync_copy(wm2_hbm, wmb)
    pltpu.sync_copy(bm2_hbm, b2b)
    bmv = [bmb[pl.ds(16 * j, 16)] for j in range(H // 16)]
    wmv = [wmb[pl.ds(16 * j, 16)] for j in range(H // 16)]
    b2s = b2b[pl.ds(0, 16)][0]
    lane = lax.iota(jnp.int32, 16)

    per_tile = E // (NC * NS)
    base = (s * NC + c) * per_tile
    nfull = per_tile // 128
    tail = per_tile - nfull * 128

    def do_chunk(b, nb, sb, db, ar, br_, ob):
        pltpu.sync_copy(src_hbm.at[pl.ds(b, nb)], sb)
        pltpu.sync_copy(dst_hbm.at[pl.ds(b, nb)], db)
        cp1 = pltpu.async_copy(a_hbm.at[sb], ar, sem1)
        cp2 = pltpu.async_copy(b_hbm.at[db], br_, sem2)
        cp1.wait()
        cp2.wait()

        def ebody(e, totv):
            acc = jnp.zeros((16,), F32)
            for j in range(H // 16):
                va = ar[e, pl.ds(16 * j, 16)]
                vb = br_[e, pl.ds(16 * j, 16)]
                acc = acc + jnp.maximum(va + vb + bmv[j], 0.0) * wmv[j]
            v = acc
            for sh in (8, 4, 2, 1):
                tbuf[pl.ds(0, 16)] = v
                v = v + tbuf[pl.ds(sh, 16)]
            tot = v[0] + b2s
            totv = jnp.where(lane == e % 16, tot, totv)
            ob[pl.ds((e // 16) * 16, 16)] = totv
            return totv

        lax.fori_loop(0, nb, ebody, jnp.zeros((16,), F32))
        pltpu.sync_copy(ob, out_hbm.at[pl.ds(b, nb)])

    def chunk(k, carry):
        do_chunk(base + k * 128, 128, srcb, dstb, arows, brows, obuf)
        return carry

    lax.fori_loop(0, nfull, chunk, 0)
    do_chunk(base + nfull * 128, tail, srcb_t, dstb_t, arows_t, brows_t, obuf_t)


RB = 1000
GRID = N // RB


def _dinv_block(dpr):
    deg = dpr[0] + dpr[1] + 1.0
    return lax.rsqrt(jnp.maximum(deg, 1.0))


def _mm_scale_body(xr, wr, br, dpr, outr):
    dinv = _dinv_block(dpr)
    h = jnp.dot(xr[...], wr[...], preferred_element_type=F32) + br[...]
    hs = h * dinv
    outr[0] = hs[:, :HH]
    outr[1] = hs[:, HH:]


def _mm_scale(x, W, b, degp, fin):
    return pl.pallas_call(
        _mm_scale_body,
        grid=(GRID,),
        in_specs=[
            pl.BlockSpec((RB, fin), lambda i: (i, 0)),
            pl.BlockSpec((fin, H), lambda i: (0, 0)),
            pl.BlockSpec((1, H), lambda i: (0, 0)),
            pl.BlockSpec((2, RB, 1), lambda i: (0, i, 0)),
        ],
        out_specs=pl.BlockSpec((2, RB, HH), lambda i: (0, i, 0)),
        out_shape=jax.ShapeDtypeStruct((2, N, HH), F32),
    )(x, W, b, degp)


def _agg_block(aggr, hsr, dpr):
    dinv = _dinv_block(dpr)
    a0 = (aggr[0] + hsr[0]) * dinv
    a1 = (aggr[1] + hsr[1]) * dinv
    return jnp.concatenate([a0, a1], axis=1)


def _stats_body(aggr, hsr, dpr, outr):
    i = pl.program_id(0)
    a = _agg_block(aggr, hsr, dpr)
    blk = jnp.stack([jnp.sum(a, axis=0), jnp.sum(a * a, axis=0)])

    @pl.when(i == 0)
    def _():
        outr[...] = jnp.zeros((2, H), F32)

    outr[...] += blk


def _stats(agg0, hs, degp):
    return pl.pallas_call(
        _stats_body,
        grid=(GRID,),
        in_specs=[
            pl.BlockSpec((2, RB, HH), lambda i: (0, i, 0)),
            pl.BlockSpec((2, RB, HH), lambda i: (0, i, 0)),
            pl.BlockSpec((2, RB, 1), lambda i: (0, i, 0)),
        ],
        out_specs=pl.BlockSpec((2, H), lambda i: (0, 0)),
        out_shape=jax.ShapeDtypeStruct((2, H), F32),
    )(agg0, hs, degp)


def _bn_relu(aggr, hsr, dpr, str_, gr, btr):
    a = _agg_block(aggr, hsr, dpr)
    mean = str_[0] * (1.0 / N)
    var = str_[1] * (1.0 / N) - mean * mean
    xn = gr[...] * (a - mean) * lax.rsqrt(var + 1e-5) + btr[...]
    return jnp.maximum(xn, 0.0)


def _bn_mm_scale_body(aggr, hsr, dpr, str_, gr, btr, wr, br, outr):
    o = _bn_relu(aggr, hsr, dpr, str_, gr, btr)
    h2 = jnp.dot(o, wr[...], preferred_element_type=F32) + br[...]
    hs2 = h2 * _dinv_block(dpr)
    outr[0] = hs2[:, :HH]
    outr[1] = hs2[:, HH:]


def _bn_mm_scale(agg0, hs, degp, stats, g, bt, W, b):
    return pl.pallas_call(
        _bn_mm_scale_body,
        grid=(GRID,),
        in_specs=[
            pl.BlockSpec((2, RB, HH), lambda i: (0, i, 0)),
            pl.BlockSpec((2, RB, HH), lambda i: (0, i, 0)),
            pl.BlockSpec((2, RB, 1), lambda i: (0, i, 0)),
            pl.BlockSpec((2, H), lambda i: (0, 0)),
            pl.BlockSpec((1, H), lambda i: (0, 0)),
            pl.BlockSpec((1, H), lambda i: (0, 0)),
            pl.BlockSpec((H, H), lambda i: (0, 0)),
            pl.BlockSpec((1, H), lambda i: (0, 0)),
        ],
        out_specs=pl.BlockSpec((2, RB, HH), lambda i: (0, i, 0)),
        out_shape=jax.ShapeDtypeStruct((2, N, HH), F32),
    )(agg0, hs, degp, stats, g, bt, W, b)


def _bn_ab_body(aggr, hsr, dpr, str_, gr, btr, war, wbr, outa, outb):
    h2 = _bn_relu(aggr, hsr, dpr, str_, gr, btr)
    outa[...] = jnp.dot(h2, war[...], preferred_element_type=F32)
    outb[...] = jnp.dot(h2, wbr[...], preferred_element_type=F32)


def _bn_ab(agg0, hs, degp, stats, g, bt, Wa, Wb):
    return pl.pallas_call(
        _bn_ab_body,
        grid=(GRID,),
        in_specs=[
            pl.BlockSpec((2, RB, HH), lambda i: (0, i, 0)),
            pl.BlockSpec((2, RB, HH), lambda i: (0, i, 0)),
            pl.BlockSpec((2, RB, 1), lambda i: (0, i, 0)),
            pl.BlockSpec((2, H), lambda i: (0, 0)),
            pl.BlockSpec((1, H), lambda i: (0, 0)),
            pl.BlockSpec((1, H), lambda i: (0, 0)),
            pl.BlockSpec((H, H), lambda i: (0, 0)),
            pl.BlockSpec((H, H), lambda i: (0, 0)),
        ],
        out_specs=[
            pl.BlockSpec((RB, H), lambda i: (i, 0)),
            pl.BlockSpec((RB, H), lambda i: (i, 0)),
        ],
        out_shape=[
            jax.ShapeDtypeStruct((N, H), F32),
            jax.ShapeDtypeStruct((N, H), F32),
        ],
    )(agg0, hs, degp, stats, g, bt, Wa, Wb)


def kernel(x, edge_index, W1, b1, g1, bt1, W2, b2, g2, bt2, Wm1, bm1, Wm2, bm2):
    src = edge_index[0]
    dst = edge_index[1]
    b1r, g1r, bt1r = b1[None, :], g1[None, :], bt1[None, :]
    b2r, g2r, bt2r = b2[None, :], g2[None, :], bt2[None, :]

    degf = _deg_kernel(dst)
    degp = degf.reshape(NC, NPAD)[:, :N].reshape(NC, N, 1)

    hs1 = _mm_scale(x, W1, b1r, degp, 128)
    agg1 = _agg_kernel(hs1.reshape(NC * N, HH), src, dst)
    agg1 = agg1.reshape(NC, N, HH)
    st1 = _stats(agg1, hs1, degp)
    hs2 = _bn_mm_scale(agg1, hs1, degp, st1, g1r, bt1r, W2, b2r)

    agg2 = _agg_kernel(hs2.reshape(NC * N, HH), src, dst)
    agg2 = agg2.reshape(NC, N, HH)
    st2 = _stats(agg2, hs2, degp)
    A, B = _bn_ab(agg2, hs2, degp, st2, g2r, bt2r, Wm1[:H], Wm1[H:])

    wm2 = Wm2[:, 0]
    bm2b = jnp.broadcast_to(bm2, (16,))
    out = _edge_kernel(A, B, src, dst, bm1, wm2, bm2b)
    return out.reshape(E, 1)


# speedup vs baseline: 8.3214x; 1.2295x over previous
import functools

import jax
import jax.numpy as jnp
from jax import lax
from jax.experimental import pallas as pl
from jax.experimental.pallas import tpu as pltpu
from jax.experimental.pallas import tpu_sc as plsc

N = 10000
E = 320000
H = 256
HH = 128
NC, NS = 2, 16
NPAD = 10240
STRIPE = NPAD // NS

F32 = jnp.float32

_mesh = plsc.VectorSubcoreMesh(core_axis_name="c", subcore_axis_name="s")


@functools.partial(
    pl.kernel,
    out_type=jax.ShapeDtypeStruct((NC * NPAD,), F32),
    mesh=_mesh,
    scratch_types=[
        pltpu.VMEM((128,), jnp.int32),
        pltpu.VMEM((128,), F32),
        pltpu.VMEM((16,), jnp.int32),
        pltpu.VMEM((16,), F32),
        pltpu.VMEM((STRIPE,), F32),
        pltpu.VMEM_SHARED((NPAD,), F32),
    ],
)
def _deg_kernel(dst_hbm, out_hbm, dstb, onesb, dstb_t, onesb_t, stage, degsp):
    c = lax.axis_index("c")
    s = lax.axis_index("s")
    for j in range(8):
        onesb[pl.ds(16 * j, 16)] = jnp.full((16,), 1.0, F32)
    onesb_t[pl.ds(0, 16)] = jnp.full((16,), 1.0, F32)
    for j in range(STRIPE // 16):
        stage[pl.ds(16 * j, 16)] = jnp.zeros((16,), F32)
    pltpu.sync_copy(stage, degsp.at[pl.ds(s * STRIPE, STRIPE)])
    plsc.subcore_barrier()

    per_tile = E // (NC * NS)
    base = (s * NC + c) * per_tile
    nfull = per_tile // 128
    tail = per_tile - nfull * 128

    def chunk(k, carry):
        b = base + k * 128
        pltpu.sync_copy(dst_hbm.at[pl.ds(b, 128)], dstb)
        pltpu.sync_copy(onesb, degsp.at[dstb], add=True)
        return carry

    lax.fori_loop(0, nfull, chunk, 0)
    bt = base + nfull * 128
    pltpu.sync_copy(dst_hbm.at[pl.ds(bt, tail)], dstb_t)
    pltpu.sync_copy(onesb_t, degsp.at[dstb_t], add=True)
    plsc.subcore_barrier()

    pltpu.sync_copy(degsp.at[pl.ds(s * STRIPE, STRIPE)], stage)
    pltpu.sync_copy(stage, out_hbm.at[pl.ds(c * NPAD + s * STRIPE, STRIPE)])


@functools.partial(
    pl.kernel,
    out_type=jax.ShapeDtypeStruct((NC * N, HH), F32),
    mesh=_mesh,
    scratch_types=[
        pltpu.VMEM((128,), jnp.int32),
        pltpu.VMEM((128,), jnp.int32),
        pltpu.VMEM((128,), jnp.int32),
        pltpu.VMEM((128, HH), F32),
        pltpu.VMEM((128, HH), F32),
        pltpu.VMEM((32,), jnp.int32),
        pltpu.VMEM((32,), jnp.int32),
        pltpu.VMEM((32,), jnp.int32),
        pltpu.VMEM((32, HH), F32),
        pltpu.VMEM_SHARED((NPAD, HH), F32),
        pltpu.SemaphoreType.DMA,
    ],
)
def _agg_kernel(hs_hbm, src_hbm, dst_hbm, out_hbm,
                srcb, dstb, idxb, rows, zb,
                srcb_t, dstb_t, idxb_t, rows_t, aggsp, sem):
    c = lax.axis_index("c")
    s = lax.axis_index("s")
    rowoff = c * N

    def zbody(i, carry):
        for j in range(HH // 16):
            zb[i, pl.ds(16 * j, 16)] = jnp.zeros((16,), F32)
        return carry

    lax.fori_loop(0, 128, zbody, 0)
    for b5 in range(STRIPE // 128):
        pltpu.sync_copy(zb, aggsp.at[pl.ds(s * STRIPE + b5 * 128, 128)])
    plsc.subcore_barrier()

    per_tile = E // NS
    ebase = s * per_tile
    nfull = per_tile // 128
    tail = per_tile - nfull * 128

    def chunk(k, carry):
        b = ebase + k * 128
        pltpu.sync_copy(src_hbm.at[pl.ds(b, 128)], srcb)
        pltpu.sync_copy(dst_hbm.at[pl.ds(b, 128)], dstb)
        for j in range(8):
            idxb[pl.ds(16 * j, 16)] = srcb[pl.ds(16 * j, 16)] + rowoff
        pltpu.async_copy(hs_hbm.at[idxb], rows, sem).wait()
        pltpu.sync_copy(rows, aggsp.at[dstb], add=True)
        return carry

    lax.fori_loop(0, nfull, chunk, 0)
    bt = ebase + nfull * 128
    pltpu.sync_copy(src_hbm.at[pl.ds(bt, tail)], srcb_t)
    pltpu.sync_copy(dst_hbm.at[pl.ds(bt, tail)], dstb_t)
    for j in range(tail // 16):
        idxb_t[pl.ds(16 * j, 16)] = srcb_t[pl.ds(16 * j, 16)] + rowoff
    pltpu.async_copy(hs_hbm.at[idxb_t], rows_t, sem).wait()
    pltpu.sync_copy(rows_t, aggsp.at[dstb_t], add=True)
    plsc.subcore_barrier()

    @pl.when(s < NS - 1)
    def _():
        for b5 in range(STRIPE // 128):
            pltpu.sync_copy(aggsp.at[pl.ds(s * STRIPE + b5 * 128, 128)], zb)
            pltpu.sync_copy(zb, out_hbm.at[pl.ds(rowoff + s * STRIPE + b5 * 128, 128)])

    @pl.when(s == NS - 1)
    def _():
        lastbase = (NS - 1) * STRIPE
        for b5 in range(3):
            pltpu.sync_copy(aggsp.at[pl.ds(lastbase + b5 * 128, 128)], zb)
            pltpu.sync_copy(zb, out_hbm.at[pl.ds(rowoff + lastbase + b5 * 128, 128)])
        pltpu.sync_copy(aggsp.at[pl.ds(lastbase + 384, 16)], rows_t.at[pl.ds(0, 16)])
        pltpu.sync_copy(rows_t.at[pl.ds(0, 16)], out_hbm.at[pl.ds(rowoff + lastbase + 384, 16)])


PT_E = E // (NC * NS)
ECH = 64
NCH = PT_E // ECH
ETAIL = PT_E - NCH * ECH

BF16 = jnp.bfloat16


@functools.partial(
    pl.kernel,
    out_type=jax.ShapeDtypeStruct((E,), F32),
    mesh=_mesh,
    scratch_types=[
        pltpu.VMEM((PT_E + 48,), jnp.int32),
        pltpu.VMEM((PT_E + 48,), jnp.int32),
        pltpu.VMEM((ECH, H), F32),
        pltpu.VMEM((ECH, H), F32),
        pltpu.VMEM((ECH, H), F32),
        pltpu.VMEM((ECH, H), F32),
        pltpu.VMEM((16, 32), F32),
        pltpu.VMEM((PT_E + 48,), F32),
        pltpu.VMEM((H,), F32),
        pltpu.VMEM((16,), F32),
        pltpu.SemaphoreType.DMA,
        pltpu.SemaphoreType.DMA,
        pltpu.SemaphoreType.DMA,
        pltpu.SemaphoreType.DMA,
    ],
)
def _edge_kernel(a_hbm, b_hbm, src_hbm, dst_hbm, wm2_hbm, bm2_hbm,
                 out_hbm, srcall, dstall, arows0, brows0, arows1, brows1,
                 accmat, outslab, wmb, b2b, semA0, semB0, semA1, semB1):
    c = lax.axis_index("c")
    s = lax.axis_index("s")
    base = (s * NC + c) * PT_E
    pltpu.sync_copy(wm2_hbm, wmb)
    pltpu.sync_copy(bm2_hbm, b2b)
    pltpu.sync_copy(src_hbm.at[pl.ds(base, PT_E)], srcall.at[pl.ds(0, PT_E)])
    pltpu.sync_copy(dst_hbm.at[pl.ds(base, PT_E)], dstall.at[pl.ds(0, PT_E)])
    wmv = [wmb[pl.ds(16 * j, 16)] for j in range(H // 16)]
    b2v = b2b[pl.ds(0, 16)]
    lane = lax.iota(jnp.int32, 16)

    bufs = ((arows0, brows0, semA0, semB0), (arows1, brows1, semA1, semB1))

    def issue(k, p):
        ar, br_, sa, sb_ = bufs[p]
        cpA = pltpu.async_copy(a_hbm.at[srcall.at[pl.ds(k * ECH, ECH)]], ar, sa)
        cpB = pltpu.async_copy(b_hbm.at[dstall.at[pl.ds(k * ECH, ECH)]], br_, sb_)
        return cpA, cpB

    def wait(p, nrows):
        ar, br_, sa, sb_ = bufs[p]
        pltpu.make_async_copy(a_hbm.at[pl.ds(0, nrows)], ar.at[pl.ds(0, nrows)], sa).wait()
        pltpu.make_async_copy(b_hbm.at[pl.ds(0, nrows)], br_.at[pl.ds(0, nrows)], sb_).wait()

    def compute_group(ar, br_, ebase16, nrows):
        zero = jnp.zeros((16,), F32)
        totv = jnp.zeros((16,), F32)
        for e in range(16):
            acc0 = jnp.zeros((16,), F32)
            acc1 = jnp.zeros((16,), F32)
            row = ebase16 + e
            for j in range(0, H // 16, 2):
                va0 = ar[row, pl.ds(16 * j, 16)]
                vb0 = br_[row, pl.ds(16 * j, 16)]
                va1 = ar[row, pl.ds(16 * (j + 1), 16)]
                vb1 = br_[row, pl.ds(16 * (j + 1), 16)]
                acc0 = acc0 + jnp.maximum(va0 + vb0, zero) * wmv[j]
                acc1 = acc1 + jnp.maximum(va1 + vb1, zero) * wmv[j + 1]
            v = acc0 + acc1
            for sh in (8, 4, 2, 1):
                accmat[e, pl.ds(0, 16)] = v
                v = v + accmat[e, pl.ds(sh, 16)]
            totv = jnp.where(lane == e, v[0], totv)
        return totv + b2v

    def compute_chunk(k, p):
        ar, br_, _, _ = bufs[p]

        def gbody(g, carry):
            tot = compute_group(ar, br_, g * 16, ECH)
            outslab[pl.ds(k * ECH + g * 16, 16)] = tot
            return carry

        lax.fori_loop(0, ECH // 16, gbody, 0)

    issue(0, 0)

    def pair(m, carry):
        k0 = m * 2

        @pl.when(k0 + 1 < NCH)
        def _():
            issue(k0 + 1, 1)

        wait(0, ECH)
        compute_chunk(k0, 0)

        @pl.when(k0 + 2 < NCH)
        def _():
            issue(k0 + 2, 0)

        @pl.when(k0 + 1 < NCH)
        def _():
            wait(1, ECH)
            compute_chunk(k0 + 1, 1)

        return carry

    lax.fori_loop(0, NCH // 2, pair, 0)

    cpA = pltpu.async_copy(
        a_hbm.at[srcall.at[pl.ds(NCH * ECH, ETAIL)]], arows0.at[pl.ds(0, ETAIL)], semA0)
    cpB = pltpu.async_copy(
        b_hbm.at[dstall.at[pl.ds(NCH * ECH, ETAIL)]], brows0.at[pl.ds(0, ETAIL)], semB0)
    cpA.wait()
    cpB.wait()
    tot = compute_group(arows0, brows0, 0, ETAIL)
    outslab[pl.ds(NCH * ECH, 16)] = tot

    pltpu.sync_copy(outslab.at[pl.ds(0, PT_E)], out_hbm.at[pl.ds(base, PT_E)])


RB = 1000
GRID = N // RB


def _dinv_block(dpr):
    deg = dpr[0] + dpr[1] + 1.0
    return lax.rsqrt(jnp.maximum(deg, 1.0))


def _mm_scale_body(xr, wr, br, dpr, outr):
    dinv = _dinv_block(dpr)
    h = jnp.dot(xr[...], wr[...], preferred_element_type=F32) + br[...]
    hs = h * dinv
    outr[0] = hs[:, :HH]
    outr[1] = hs[:, HH:]


def _mm_scale(x, W, b, degp, fin):
    return pl.pallas_call(
        _mm_scale_body,
        grid=(GRID,),
        in_specs=[
            pl.BlockSpec((RB, fin), lambda i: (i, 0)),
            pl.BlockSpec((fin, H), lambda i: (0, 0)),
            pl.BlockSpec((1, H), lambda i: (0, 0)),
            pl.BlockSpec((2, RB, 1), lambda i: (0, i, 0)),
        ],
        out_specs=pl.BlockSpec((2, RB, HH), lambda i: (0, i, 0)),
        out_shape=jax.ShapeDtypeStruct((2, N, HH), F32),
    )(x, W, b, degp)


def _agg_block(aggr, hsr, dpr):
    dinv = _dinv_block(dpr)
    a0 = (aggr[0] + hsr[0]) * dinv
    a1 = (aggr[1] + hsr[1]) * dinv
    return jnp.concatenate([a0, a1], axis=1)


def _stats_body(aggr, hsr, dpr, outr):
    i = pl.program_id(0)
    a = _agg_block(aggr, hsr, dpr)
    blk = jnp.stack([jnp.sum(a, axis=0), jnp.sum(a * a, axis=0)])

    @pl.when(i == 0)
    def _():
        outr[...] = jnp.zeros((2, H), F32)

    outr[...] += blk


def _stats(agg0, hs, degp):
    return pl.pallas_call(
        _stats_body,
        grid=(GRID,),
        in_specs=[
            pl.BlockSpec((2, RB, HH), lambda i: (0, i, 0)),
            pl.BlockSpec((2, RB, HH), lambda i: (0, i, 0)),
            pl.BlockSpec((2, RB, 1), lambda i: (0, i, 0)),
        ],
        out_specs=pl.BlockSpec((2, H), lambda i: (0, 0)),
        out_shape=jax.ShapeDtypeStruct((2, H), F32),
    )(agg0, hs, degp)


def _bn_relu(aggr, hsr, dpr, str_, gr, btr):
    a = _agg_block(aggr, hsr, dpr)
    mean = str_[0] * (1.0 / N)
    var = str_[1] * (1.0 / N) - mean * mean
    xn = gr[...] * (a - mean) * lax.rsqrt(var + 1e-5) + btr[...]
    return jnp.maximum(xn, 0.0)


def _bn_mm_scale_body(aggr, hsr, dpr, str_, gr, btr, wr, br, outr):
    o = _bn_relu(aggr, hsr, dpr, str_, gr, btr)
    h2 = jnp.dot(o, wr[...], preferred_element_type=F32) + br[...]
    hs2 = h2 * _dinv_block(dpr)
    outr[0] = hs2[:, :HH]
    outr[1] = hs2[:, HH:]


def _bn_mm_scale(agg0, hs, degp, stats, g, bt, W, b):
    return pl.pallas_call(
        _bn_mm_scale_body,
        grid=(GRID,),
        in_specs=[
            pl.BlockSpec((2, RB, HH), lambda i: (0, i, 0)),
            pl.BlockSpec((2, RB, HH), lambda i: (0, i, 0)),
            pl.BlockSpec((2, RB, 1), lambda i: (0, i, 0)),
            pl.BlockSpec((2, H), lambda i: (0, 0)),
            pl.BlockSpec((1, H), lambda i: (0, 0)),
            pl.BlockSpec((1, H), lambda i: (0, 0)),
            pl.BlockSpec((H, H), lambda i: (0, 0)),
            pl.BlockSpec((1, H), lambda i: (0, 0)),
        ],
        out_specs=pl.BlockSpec((2, RB, HH), lambda i: (0, i, 0)),
        out_shape=jax.ShapeDtypeStruct((2, N, HH), F32),
    )(agg0, hs, degp, stats, g, bt, W, b)


def _bn_ab_body(aggr, hsr, dpr, str_, gr, btr, war, wbr, bmr, outa, outb):
    h2 = _bn_relu(aggr, hsr, dpr, str_, gr, btr)
    outa[...] = jnp.dot(h2, war[...], preferred_element_type=F32) + bmr[...]
    outb[...] = jnp.dot(h2, wbr[...], preferred_element_type=F32)


def _bn_ab(agg0, hs, degp, stats, g, bt, Wa, Wb, bm):
    return pl.pallas_call(
        _bn_ab_body,
        grid=(GRID,),
        in_specs=[
            pl.BlockSpec((2, RB, HH), lambda i: (0, i, 0)),
            pl.BlockSpec((2, RB, HH), lambda i: (0, i, 0)),
            pl.BlockSpec((2, RB, 1), lambda i: (0, i, 0)),
            pl.BlockSpec((2, H), lambda i: (0, 0)),
            pl.BlockSpec((1, H), lambda i: (0, 0)),
            pl.BlockSpec((1, H), lambda i: (0, 0)),
            pl.BlockSpec((H, H), lambda i: (0, 0)),
            pl.BlockSpec((H, H), lambda i: (0, 0)),
            pl.BlockSpec((1, H), lambda i: (0, 0)),
        ],
        out_specs=[
            pl.BlockSpec((RB, H), lambda i: (i, 0)),
            pl.BlockSpec((RB, H), lambda i: (i, 0)),
        ],
        out_shape=[
            jax.ShapeDtypeStruct((N, H), F32),
            jax.ShapeDtypeStruct((N, H), F32),
        ],
    )(agg0, hs, degp, stats, g, bt, Wa, Wb, bm)


def kernel(x, edge_index, W1, b1, g1, bt1, W2, b2, g2, bt2, Wm1, bm1, Wm2, bm2):
    src = edge_index[0]
    dst = edge_index[1]
    b1r, g1r, bt1r = b1[None, :], g1[None, :], bt1[None, :]
    b2r, g2r, bt2r = b2[None, :], g2[None, :], bt2[None, :]

    degf = _deg_kernel(dst)
    degp = degf.reshape(NC, NPAD)[:, :N].reshape(NC, N, 1)

    hs1 = _mm_scale(x, W1, b1r, degp, 128)
    agg1 = _agg_kernel(hs1.reshape(NC * N, HH), src, dst)
    agg1 = agg1.reshape(NC, N, HH)
    st1 = _stats(agg1, hs1, degp)
    hs2 = _bn_mm_scale(agg1, hs1, degp, st1, g1r, bt1r, W2, b2r)

    agg2 = _agg_kernel(hs2.reshape(NC * N, HH), src, dst)
    agg2 = agg2.reshape(NC, N, HH)
    st2 = _stats(agg2, hs2, degp)
    A, B = _bn_ab(agg2, hs2, degp, st2, g2r, bt2r, Wm1[:H], Wm1[H:], bm1[None, :])

    wm2 = Wm2[:, 0]
    bm2b = jnp.broadcast_to(bm2, (16,))
    out = _edge_kernel(A, B, src, dst, wm2, bm2b)
    return out.reshape(E, 1)


# speedup vs baseline: 11.5075x; 1.3829x over previous
import functools

import jax
import jax.numpy as jnp
from jax import lax
from jax.experimental import pallas as pl
from jax.experimental.pallas import tpu as pltpu
from jax.experimental.pallas import tpu_sc as plsc

N = 10000
E = 320000
H = 256
HH = 128
NC, NS = 2, 16
NPAD = 10240
STRIPE = NPAD // NS

F32 = jnp.float32

_mesh = plsc.VectorSubcoreMesh(core_axis_name="c", subcore_axis_name="s")


@functools.partial(
    pl.kernel,
    out_type=jax.ShapeDtypeStruct((NC * NPAD,), F32),
    mesh=_mesh,
    scratch_types=[
        pltpu.VMEM((128,), jnp.int32),
        pltpu.VMEM((128,), F32),
        pltpu.VMEM((16,), jnp.int32),
        pltpu.VMEM((16,), F32),
        pltpu.VMEM((STRIPE,), F32),
        pltpu.VMEM_SHARED((NPAD,), F32),
    ],
)
def _deg_kernel(dst_hbm, out_hbm, dstb, onesb, dstb_t, onesb_t, stage, degsp):
    c = lax.axis_index("c")
    s = lax.axis_index("s")
    for j in range(8):
        onesb[pl.ds(16 * j, 16)] = jnp.full((16,), 1.0, F32)
    onesb_t[pl.ds(0, 16)] = jnp.full((16,), 1.0, F32)
    for j in range(STRIPE // 16):
        stage[pl.ds(16 * j, 16)] = jnp.zeros((16,), F32)
    pltpu.sync_copy(stage, degsp.at[pl.ds(s * STRIPE, STRIPE)])
    plsc.subcore_barrier()

    per_tile = E // (NC * NS)
    base = (s * NC + c) * per_tile
    nfull = per_tile // 128
    tail = per_tile - nfull * 128

    def chunk(k, carry):
        b = base + k * 128
        pltpu.sync_copy(dst_hbm.at[pl.ds(b, 128)], dstb)
        pltpu.sync_copy(onesb, degsp.at[dstb], add=True)
        return carry

    lax.fori_loop(0, nfull, chunk, 0)
    bt = base + nfull * 128
    pltpu.sync_copy(dst_hbm.at[pl.ds(bt, tail)], dstb_t)
    pltpu.sync_copy(onesb_t, degsp.at[dstb_t], add=True)
    plsc.subcore_barrier()

    pltpu.sync_copy(degsp.at[pl.ds(s * STRIPE, STRIPE)], stage)
    pltpu.sync_copy(stage, out_hbm.at[pl.ds(c * NPAD + s * STRIPE, STRIPE)])


ECH_A = 64
PT_A = E // NS
NCH_A = PT_A // ECH_A
ATAIL = PT_A - NCH_A * ECH_A


@functools.partial(
    pl.kernel,
    out_type=jax.ShapeDtypeStruct((NC * N, HH), F32),
    mesh=_mesh,
    scratch_types=[
        pltpu.VMEM((PT_A + 96,), jnp.int32),
        pltpu.VMEM((ECH_A,), jnp.int32),
        pltpu.VMEM((ECH_A,), jnp.int32),
        pltpu.VMEM((ECH_A, HH), F32),
        pltpu.VMEM((ECH_A, HH), F32),
        pltpu.VMEM((32,), jnp.int32),
        pltpu.VMEM((32, HH), F32),
        pltpu.VMEM_SHARED((NPAD, HH), F32),
        pltpu.SemaphoreType.DMA,
        pltpu.SemaphoreType.DMA,
        pltpu.SemaphoreType.DMA,
        pltpu.SemaphoreType.DMA,
    ],
)
def _agg_kernel(hs_hbm, src_hbm, dst_hbm, out_hbm,
                srcall, dstb0, dstb1, rows0, rows1, dstb_t, rows_t, aggsp,
                semg0, semd0, semg1, semd1):
    c = lax.axis_index("c")
    s = lax.axis_index("s")
    rowoff = c * N
    ebase = s * PT_A

    def zbody(i, carry):
        for j in range(HH // 16):
            rows0[i, pl.ds(16 * j, 16)] = jnp.zeros((16,), F32)
        return carry

    lax.fori_loop(0, ECH_A, zbody, 0)
    for b in range(STRIPE // ECH_A):
        pltpu.sync_copy(rows0, aggsp.at[pl.ds(s * STRIPE + b * ECH_A, ECH_A)])

    pltpu.sync_copy(src_hbm.at[pl.ds(ebase, PT_A)], srcall.at[pl.ds(0, PT_A)])

    @pl.when(c == 1)
    def _():
        def abody(i, carry):
            srcall[pl.ds(16 * i, 16)] = srcall[pl.ds(16 * i, 16)] + rowoff
            return carry

        lax.fori_loop(0, PT_A // 16, abody, 0)

    plsc.subcore_barrier()

    bufs = ((rows0, dstb0, semg0, semd0), (rows1, dstb1, semg1, semd1))

    def issue_in(k, p):
        rows, dstb, sg, sd = bufs[p]
        pltpu.async_copy(hs_hbm.at[srcall.at[pl.ds(k * ECH_A, ECH_A)]], rows, sg)
        pltpu.async_copy(dst_hbm.at[pl.ds(ebase + k * ECH_A, ECH_A)], dstb, sd)

    def wait_in(p):
        rows, dstb, sg, sd = bufs[p]
        pltpu.make_async_copy(hs_hbm.at[pl.ds(0, ECH_A)], rows, sg).wait()
        pltpu.make_async_copy(dst_hbm.at[pl.ds(0, ECH_A)], dstb, sd).wait()

    def scatter(p):
        rows, dstb, _, _ = bufs[p]
        pltpu.sync_copy(rows, aggsp.at[dstb], add=True)

    issue_in(0, 0)

    def pair(m, carry):
        k0 = m * 2
        issue_in(k0 + 1, 1)
        wait_in(0)
        scatter(0)

        @pl.when(k0 + 2 < NCH_A)
        def _():
            issue_in(k0 + 2, 0)

        wait_in(1)
        scatter(1)
        return carry

    lax.fori_loop(0, NCH_A // 2, pair, 0)

    bt = NCH_A * ECH_A
    cpG = pltpu.async_copy(
        hs_hbm.at[srcall.at[pl.ds(bt, ATAIL)]], rows_t, semg0)
    cpD = pltpu.async_copy(dst_hbm.at[pl.ds(ebase + bt, ATAIL)], dstb_t, semd0)
    cpG.wait()
    cpD.wait()
    pltpu.sync_copy(rows_t, aggsp.at[dstb_t], add=True)
    plsc.subcore_barrier()

    @pl.when(s < NS - 1)
    def _():
        for b in range(STRIPE // ECH_A):
            pltpu.sync_copy(aggsp.at[pl.ds(s * STRIPE + b * ECH_A, ECH_A)], rows0)
            pltpu.sync_copy(rows0, out_hbm.at[pl.ds(rowoff + s * STRIPE + b * ECH_A, ECH_A)])

    @pl.when(s == NS - 1)
    def _():
        lastbase = (NS - 1) * STRIPE
        for b in range((N - lastbase) // ECH_A):
            pltpu.sync_copy(aggsp.at[pl.ds(lastbase + b * ECH_A, ECH_A)], rows0)
            pltpu.sync_copy(rows0, out_hbm.at[pl.ds(rowoff + lastbase + b * ECH_A, ECH_A)])
        rem_base = lastbase + ((N - lastbase) // ECH_A) * ECH_A
        pltpu.sync_copy(aggsp.at[pl.ds(rem_base, 16)], rows_t.at[pl.ds(0, 16)])
        pltpu.sync_copy(rows_t.at[pl.ds(0, 16)], out_hbm.at[pl.ds(rowoff + rem_base, 16)])


PT_E = E // (NC * NS)
ECH = 64
NCH = PT_E // ECH
ETAIL = PT_E - NCH * ECH

BF16 = jnp.bfloat16


@functools.partial(
    pl.kernel,
    out_type=jax.ShapeDtypeStruct((E,), F32),
    mesh=_mesh,
    scratch_types=[
        pltpu.VMEM((PT_E + 48,), jnp.int32),
        pltpu.VMEM((PT_E + 48,), jnp.int32),
        pltpu.VMEM((ECH, H), F32),
        pltpu.VMEM((ECH, H), F32),
        pltpu.VMEM((ECH, H), F32),
        pltpu.VMEM((ECH, H), F32),
        pltpu.VMEM((16, 32), F32),
        pltpu.VMEM((PT_E + 48,), F32),
        pltpu.VMEM((H,), F32),
        pltpu.VMEM((16,), F32),
        pltpu.SemaphoreType.DMA,
        pltpu.SemaphoreType.DMA,
        pltpu.SemaphoreType.DMA,
        pltpu.SemaphoreType.DMA,
    ],
)
def _edge_kernel(a_hbm, b_hbm, src_hbm, dst_hbm, wm2_hbm, bm2_hbm,
                 out_hbm, srcall, dstall, arows0, brows0, arows1, brows1,
                 accmat, outslab, wmb, b2b, semA0, semB0, semA1, semB1):
    c = lax.axis_index("c")
    s = lax.axis_index("s")
    base = (s * NC + c) * PT_E
    pltpu.sync_copy(wm2_hbm, wmb)
    pltpu.sync_copy(bm2_hbm, b2b)
    pltpu.sync_copy(src_hbm.at[pl.ds(base, PT_E)], srcall.at[pl.ds(0, PT_E)])
    pltpu.sync_copy(dst_hbm.at[pl.ds(base, PT_E)], dstall.at[pl.ds(0, PT_E)])
    wmv = [wmb[pl.ds(16 * j, 16)] for j in range(H // 16)]
    b2v = b2b[pl.ds(0, 16)]
    lane = lax.iota(jnp.int32, 16)

    bufs = ((arows0, brows0, semA0, semB0), (arows1, brows1, semA1, semB1))

    def issue(k, p):
        ar, br_, sa, sb_ = bufs[p]
        cpA = pltpu.async_copy(a_hbm.at[srcall.at[pl.ds(k * ECH, ECH)]], ar, sa)
        cpB = pltpu.async_copy(b_hbm.at[dstall.at[pl.ds(k * ECH, ECH)]], br_, sb_)
        return cpA, cpB

    def wait(p, nrows):
        ar, br_, sa, sb_ = bufs[p]
        pltpu.make_async_copy(a_hbm.at[pl.ds(0, nrows)], ar.at[pl.ds(0, nrows)], sa).wait()
        pltpu.make_async_copy(b_hbm.at[pl.ds(0, nrows)], br_.at[pl.ds(0, nrows)], sb_).wait()

    def compute_group(ar, br_, ebase16, nrows):
        zero = jnp.zeros((16,), F32)
        totv = jnp.zeros((16,), F32)
        for e in range(16):
            acc0 = jnp.zeros((16,), F32)
            acc1 = jnp.zeros((16,), F32)
            row = ebase16 + e
            for j in range(0, H // 16, 2):
                va0 = ar[row, pl.ds(16 * j, 16)]
                vb0 = br_[row, pl.ds(16 * j, 16)]
                va1 = ar[row, pl.ds(16 * (j + 1), 16)]
                vb1 = br_[row, pl.ds(16 * (j + 1), 16)]
                acc0 = acc0 + jnp.maximum(va0 + vb0, zero) * wmv[j]
                acc1 = acc1 + jnp.maximum(va1 + vb1, zero) * wmv[j + 1]
            v = acc0 + acc1
            for sh in (8, 4, 2, 1):
                accmat[e, pl.ds(0, 16)] = v
                v = v + accmat[e, pl.ds(sh, 16)]
            totv = jnp.where(lane == e, v[0], totv)
        return totv + b2v

    def compute_chunk(k, p):
        ar, br_, _, _ = bufs[p]

        def gbody(g, carry):
            tot = compute_group(ar, br_, g * 16, ECH)
            outslab[pl.ds(k * ECH + g * 16, 16)] = tot
            return carry

        lax.fori_loop(0, ECH // 16, gbody, 0)

    issue(0, 0)

    def pair(m, carry):
        k0 = m * 2

        @pl.when(k0 + 1 < NCH)
        def _():
            issue(k0 + 1, 1)

        wait(0, ECH)
        compute_chunk(k0, 0)

        @pl.when(k0 + 2 < NCH)
        def _():
            issue(k0 + 2, 0)

        @pl.when(k0 + 1 < NCH)
        def _():
            wait(1, ECH)
            compute_chunk(k0 + 1, 1)

        return carry

    lax.fori_loop(0, NCH // 2, pair, 0)

    cpA = pltpu.async_copy(
        a_hbm.at[srcall.at[pl.ds(NCH * ECH, ETAIL)]], arows0.at[pl.ds(0, ETAIL)], semA0)
    cpB = pltpu.async_copy(
        b_hbm.at[dstall.at[pl.ds(NCH * ECH, ETAIL)]], brows0.at[pl.ds(0, ETAIL)], semB0)
    cpA.wait()
    cpB.wait()
    tot = compute_group(arows0, brows0, 0, ETAIL)
    outslab[pl.ds(NCH * ECH, 16)] = tot

    pltpu.sync_copy(outslab.at[pl.ds(0, PT_E)], out_hbm.at[pl.ds(base, PT_E)])


RB = 1000
GRID = N // RB


def _dinv_block(dpr):
    deg = dpr[0] + dpr[1] + 1.0
    return lax.rsqrt(jnp.maximum(deg, 1.0))


def _mm_scale_body(xr, wr, br, dpr, outr):
    dinv = _dinv_block(dpr)
    h = jnp.dot(xr[...], wr[...], preferred_element_type=F32) + br[...]
    hs = h * dinv
    outr[0] = hs[:, :HH]
    outr[1] = hs[:, HH:]


def _mm_scale(x, W, b, degp, fin):
    return pl.pallas_call(
        _mm_scale_body,
        grid=(GRID,),
        in_specs=[
            pl.BlockSpec((RB, fin), lambda i: (i, 0)),
            pl.BlockSpec((fin, H), lambda i: (0, 0)),
            pl.BlockSpec((1, H), lambda i: (0, 0)),
            pl.BlockSpec((2, RB, 1), lambda i: (0, i, 0)),
        ],
        out_specs=pl.BlockSpec((2, RB, HH), lambda i: (0, i, 0)),
        out_shape=jax.ShapeDtypeStruct((2, N, HH), F32),
    )(x, W, b, degp)


def _agg_block(aggr, hsr, dpr):
    dinv = _dinv_block(dpr)
    a0 = (aggr[0] + hsr[0]) * dinv
    a1 = (aggr[1] + hsr[1]) * dinv
    return jnp.concatenate([a0, a1], axis=1)


def _stats_body(aggr, hsr, dpr, outr):
    i = pl.program_id(0)
    a = _agg_block(aggr, hsr, dpr)
    blk = jnp.stack([jnp.sum(a, axis=0), jnp.sum(a * a, axis=0)])

    @pl.when(i == 0)
    def _():
        outr[...] = jnp.zeros((2, H), F32)

    outr[...] += blk


def _stats(agg0, hs, degp):
    return pl.pallas_call(
        _stats_body,
        grid=(GRID,),
        in_specs=[
            pl.BlockSpec((2, RB, HH), lambda i: (0, i, 0)),
            pl.BlockSpec((2, RB, HH), lambda i: (0, i, 0)),
            pl.BlockSpec((2, RB, 1), lambda i: (0, i, 0)),
        ],
        out_specs=pl.BlockSpec((2, H), lambda i: (0, 0)),
        out_shape=jax.ShapeDtypeStruct((2, H), F32),
    )(agg0, hs, degp)


def _bn_relu(aggr, hsr, dpr, str_, gr, btr):
    a = _agg_block(aggr, hsr, dpr)
    mean = str_[0] * (1.0 / N)
    var = str_[1] * (1.0 / N) - mean * mean
    xn = gr[...] * (a - mean) * lax.rsqrt(var + 1e-5) + btr[...]
    return jnp.maximum(xn, 0.0)


def _bn_mm_scale_body(aggr, hsr, dpr, str_, gr, btr, wr, br, outr):
    o = _bn_relu(aggr, hsr, dpr, str_, gr, btr)
    h2 = jnp.dot(o, wr[...], preferred_element_type=F32) + br[...]
    hs2 = h2 * _dinv_block(dpr)
    outr[0] = hs2[:, :HH]
    outr[1] = hs2[:, HH:]


def _bn_mm_scale(agg0, hs, degp, stats, g, bt, W, b):
    return pl.pallas_call(
        _bn_mm_scale_body,
        grid=(GRID,),
        in_specs=[
            pl.BlockSpec((2, RB, HH), lambda i: (0, i, 0)),
            pl.BlockSpec((2, RB, HH), lambda i: (0, i, 0)),
            pl.BlockSpec((2, RB, 1), lambda i: (0, i, 0)),
            pl.BlockSpec((2, H), lambda i: (0, 0)),
            pl.BlockSpec((1, H), lambda i: (0, 0)),
            pl.BlockSpec((1, H), lambda i: (0, 0)),
            pl.BlockSpec((H, H), lambda i: (0, 0)),
            pl.BlockSpec((1, H), lambda i: (0, 0)),
        ],
        out_specs=pl.BlockSpec((2, RB, HH), lambda i: (0, i, 0)),
        out_shape=jax.ShapeDtypeStruct((2, N, HH), F32),
    )(agg0, hs, degp, stats, g, bt, W, b)


def _bn_ab_body(aggr, hsr, dpr, str_, gr, btr, war, wbr, bmr, outa, outb):
    h2 = _bn_relu(aggr, hsr, dpr, str_, gr, btr)
    outa[...] = jnp.dot(h2, war[...], preferred_element_type=F32) + bmr[...]
    outb[...] = jnp.dot(h2, wbr[...], preferred_element_type=F32)


def _bn_ab(agg0, hs, degp, stats, g, bt, Wa, Wb, bm):
    return pl.pallas_call(
        _bn_ab_body,
        grid=(GRID,),
        in_specs=[
            pl.BlockSpec((2, RB, HH), lambda i: (0, i, 0)),
            pl.BlockSpec((2, RB, HH), lambda i: (0, i, 0)),
            pl.BlockSpec((2, RB, 1), lambda i: (0, i, 0)),
            pl.BlockSpec((2, H), lambda i: (0, 0)),
            pl.BlockSpec((1, H), lambda i: (0, 0)),
            pl.BlockSpec((1, H), lambda i: (0, 0)),
            pl.BlockSpec((H, H), lambda i: (0, 0)),
            pl.BlockSpec((H, H), lambda i: (0, 0)),
            pl.BlockSpec((1, H), lambda i: (0, 0)),
        ],
        out_specs=[
            pl.BlockSpec((RB, H), lambda i: (i, 0)),
            pl.BlockSpec((RB, H), lambda i: (i, 0)),
        ],
        out_shape=[
            jax.ShapeDtypeStruct((N, H), F32),
            jax.ShapeDtypeStruct((N, H), F32),
        ],
    )(agg0, hs, degp, stats, g, bt, Wa, Wb, bm)


def kernel(x, edge_index, W1, b1, g1, bt1, W2, b2, g2, bt2, Wm1, bm1, Wm2, bm2):
    src = edge_index[0]
    dst = edge_index[1]
    b1r, g1r, bt1r = b1[None, :], g1[None, :], bt1[None, :]
    b2r, g2r, bt2r = b2[None, :], g2[None, :], bt2[None, :]

    degf = _deg_kernel(dst)
    degp = degf.reshape(NC, NPAD)[:, :N].reshape(NC, N, 1)

    hs1 = _mm_scale(x, W1, b1r, degp, 128)
    agg1 = _agg_kernel(hs1.reshape(NC * N, HH), src, dst)
    agg1 = agg1.reshape(NC, N, HH)
    st1 = _stats(agg1, hs1, degp)
    hs2 = _bn_mm_scale(agg1, hs1, degp, st1, g1r, bt1r, W2, b2r)

    agg2 = _agg_kernel(hs2.reshape(NC * N, HH), src, dst)
    agg2 = agg2.reshape(NC, N, HH)
    st2 = _stats(agg2, hs2, degp)
    A, B = _bn_ab(agg2, hs2, degp, st2, g2r, bt2r, Wm1[:H], Wm1[H:], bm1[None, :])

    wm2 = Wm2[:, 0]
    bm2b = jnp.broadcast_to(bm2, (16,))
    out = _edge_kernel(A, B, src, dst, wm2, bm2b)
    return out.reshape(E, 1)
